# baseline (speedup 1.0000x reference)
import jax
import jax.numpy as jnp
from jax.experimental import pallas as pl

N = 10000
NFEAT = 128
NHID = 128
NCLASS = 64
BM = 200


def _s1_body(x_ref, w1_ref, o_ref):
    o_ref[...] = jnp.dot(x_ref[...], w1_ref[...],
                         preferred_element_type=jnp.float32)


def _main_body(adj_ref, s1_ref, b1_ref, w2_ref, yt_ref):
    i = pl.program_id(0)
    s1 = s1_ref[...].astype(jnp.bfloat16)
    w2 = w2_ref[...].astype(jnp.bfloat16)

    def chain(a32):
        a = a32.astype(jnp.bfloat16)
        h = jnp.dot(a, s1, preferred_element_type=jnp.float32)
        h = jnp.maximum(h + b1_ref[...], 0.0)
        g = jnp.dot(h.astype(jnp.bfloat16), w2,
                    preferred_element_type=jnp.float32)
        return jax.lax.dot_general(
            g.astype(jnp.bfloat16), a, (((0,), (0,)), ((), ())),
            preferred_element_type=jnp.float32)

    half = BM // 2
    contrib = (chain(adj_ref[:half, :]) + chain(adj_ref[half:, :]))

    @pl.when(i == 0)
    def _():
        yt_ref[...] = contrib

    @pl.when(i > 0)
    def _():
        yt_ref[...] += contrib


def _epilogue_body(yt_ref, b2_ref, o_ref):
    o = yt_ref[...].T + b2_ref[...]
    m = jnp.max(o, axis=1, keepdims=True)
    lse = m + jnp.log(jnp.sum(jnp.exp(o - m), axis=1, keepdims=True))
    o_ref[...] = o - lse


@jax.jit
def kernel(x, adj, W1, b1, W2, b2):
    grid = N // BM
    s1 = pl.pallas_call(
        _s1_body,
        grid=(N // 2000,),
        in_specs=[
            pl.BlockSpec((2000, NFEAT), lambda i: (i, 0)),
            pl.BlockSpec((NFEAT, NHID), lambda i: (0, 0)),
        ],
        out_specs=pl.BlockSpec((2000, NHID), lambda i: (i, 0)),
        out_shape=jax.ShapeDtypeStruct((N, NHID), jnp.float32),
    )(x, W1)

    yt = pl.pallas_call(
        _main_body,
        grid=(grid,),
        in_specs=[
            pl.BlockSpec((BM, N), lambda i: (i, 0)),
            pl.BlockSpec((N, NHID), lambda i: (0, 0)),
            pl.BlockSpec((1, NHID), lambda i: (0, 0)),
            pl.BlockSpec((NHID, NCLASS), lambda i: (0, 0)),
        ],
        out_specs=pl.BlockSpec((NCLASS, N), lambda i: (0, 0)),
        out_shape=jax.ShapeDtypeStruct((NCLASS, N), jnp.float32),
    )(adj, s1, b1.reshape(1, NHID), W2)

    out = pl.pallas_call(
        _epilogue_body,
        grid=(1,),
        in_specs=[
            pl.BlockSpec((NCLASS, N), lambda i: (0, 0)),
            pl.BlockSpec((1, NCLASS), lambda i: (0, 0)),
        ],
        out_specs=pl.BlockSpec((N, NCLASS), lambda i: (0, 0)),
        out_shape=jax.ShapeDtypeStruct((N, NCLASS), jnp.float32),
    )(yt, b2.reshape(1, NCLASS))
    return out


# speedup vs baseline: 1.1440x; 1.1440x over previous
import jax
import jax.numpy as jnp
from jax.experimental import pallas as pl

N = 10000
NFEAT = 128
NHID = 128
NCLASS = 64
BM = 400


def _s1_body(x_ref, w1_ref, o_ref):
    o_ref[...] = jnp.dot(x_ref[...], w1_ref[...],
                         preferred_element_type=jnp.float32)


def _main_body(adj0_ref, adj1_ref, s1_ref, b1_ref, w2_ref, yt_ref):
    i = pl.program_id(0)
    s1 = s1_ref[...].astype(jnp.bfloat16)
    w2 = w2_ref[...].astype(jnp.bfloat16)

    def chain(a32):
        a = a32.astype(jnp.bfloat16)
        h = jnp.dot(a, s1, preferred_element_type=jnp.float32)
        h = jnp.maximum(h + b1_ref[...], 0.0)
        g = jnp.dot(h.astype(jnp.bfloat16), w2,
                    preferred_element_type=jnp.float32)
        return jax.lax.dot_general(
            g.astype(jnp.bfloat16), a, (((0,), (0,)), ((), ())),
            preferred_element_type=jnp.float32)

    contrib = chain(adj0_ref[...]) + chain(adj1_ref[...])

    @pl.when(i == 0)
    def _():
        yt_ref[...] = contrib

    @pl.when(i > 0)
    def _():
        yt_ref[...] += contrib


def _epilogue_body(yt_ref, b2_ref, o_ref):
    o = yt_ref[...].T + b2_ref[...]
    m = jnp.max(o, axis=1, keepdims=True)
    lse = m + jnp.log(jnp.sum(jnp.exp(o - m), axis=1, keepdims=True))
    o_ref[...] = o - lse


@jax.jit
def kernel(x, adj, W1, b1, W2, b2):
    grid = N // BM
    s1 = pl.pallas_call(
        _s1_body,
        grid=(N // 2000,),
        in_specs=[
            pl.BlockSpec((2000, NFEAT), lambda i: (i, 0)),
            pl.BlockSpec((NFEAT, NHID), lambda i: (0, 0)),
        ],
        out_specs=pl.BlockSpec((2000, NHID), lambda i: (i, 0)),
        out_shape=jax.ShapeDtypeStruct((N, NHID), jnp.float32),
    )(x, W1)

    yt = pl.pallas_call(
        _main_body,
        grid=(grid,),
        in_specs=[
            pl.BlockSpec((BM // 2, N), lambda i: (2 * i, 0)),
            pl.BlockSpec((BM // 2, N), lambda i: (2 * i + 1, 0)),
            pl.BlockSpec((N, NHID), lambda i: (0, 0)),
            pl.BlockSpec((1, NHID), lambda i: (0, 0)),
            pl.BlockSpec((NHID, NCLASS), lambda i: (0, 0)),
        ],
        out_specs=pl.BlockSpec((NCLASS, N), lambda i: (0, 0)),
        out_shape=jax.ShapeDtypeStruct((NCLASS, N), jnp.float32),
    )(adj, adj, s1, b1.reshape(1, NHID), W2)

    out = pl.pallas_call(
        _epilogue_body,
        grid=(1,),
        in_specs=[
            pl.BlockSpec((NCLASS, N), lambda i: (0, 0)),
            pl.BlockSpec((1, NCLASS), lambda i: (0, 0)),
        ],
        out_specs=pl.BlockSpec((N, NCLASS), lambda i: (0, 0)),
        out_shape=jax.ShapeDtypeStruct((N, NCLASS), jnp.float32),
    )(yt, b2.reshape(1, NCLASS))
    return out
